# Initial kernel scaffold; baseline (speedup 1.0000x reference)
import jax
import jax.numpy as jnp
from jax.experimental import pallas as pl


def kernel(sp_structure, geo_feat_M, sem_feat_M, W_geo, ln_g, ln_b, W_bdy, b_bdy, Wq, Wk, Wv, pos_emb, W_out, b_out, W_cls, b_cls):
    raise NotImplementedError("write your pallas kernel here")


# speedup vs baseline: 1.0077x; 1.0077x over previous
import functools

import jax
import jax.numpy as jnp
from jax.experimental import pallas as pl
from jax.experimental.pallas import tpu as pltpu

R = 1
K_SEQ = 27
DIAM = 3
ATTN = 64
GEO = 32
SEM = 32
NCLS = 13
M = 20000


def _dense_body(geo_ref, sem_ref, Wg_ref, g_ref, b_ref, Wb_ref, bb_ref,
                Wq_ref, Wk_ref, Wv_ref,
                qgeo_ref, bdy_ref, qp_ref, kw_ref, vw_ref):
    x = geo_ref[...] @ Wg_ref[...]
    mu = jnp.mean(x, axis=-1, keepdims=True)
    var = jnp.mean((x - mu) ** 2, axis=-1, keepdims=True)
    q = (x - mu) / jnp.sqrt(var + 1e-5) * g_ref[...] + b_ref[...]
    q = jnp.maximum(q, 0.0)
    qgeo_ref[...] = q
    bdy_ref[...] = q @ Wb_ref[...] + bb_ref[...]
    qp_ref[...] = q @ Wq_ref[...]
    kw_ref[...] = q @ Wk_ref[...]
    vw_ref[...] = sem_ref[...] @ Wv_ref[...]


def _dense_precompute(geo, sem, W_geo, ln_g, ln_b, W_bdy, b_bdy, Wq, Wk, Wv):
    B = 2000
    grid = (M // B,)
    bs_row = lambda d: pl.BlockSpec((B, d), lambda i: (i, 0))
    bs_full = lambda a, b: pl.BlockSpec((a, b), lambda i: (0, 0))
    out_shapes = (
        jax.ShapeDtypeStruct((M, ATTN), jnp.float32),
        jax.ShapeDtypeStruct((M, 1), jnp.float32),
        jax.ShapeDtypeStruct((M, ATTN), jnp.float32),
        jax.ShapeDtypeStruct((M, ATTN), jnp.float32),
        jax.ShapeDtypeStruct((M, ATTN), jnp.float32),
    )
    return pl.pallas_call(
        _dense_body,
        grid=grid,
        in_specs=[
            bs_row(GEO), bs_row(SEM),
            bs_full(GEO, ATTN), bs_full(1, ATTN), bs_full(1, ATTN),
            bs_full(ATTN, 1), bs_full(1, 1),
            bs_full(ATTN, ATTN), bs_full(ATTN, ATTN), bs_full(SEM, ATTN),
        ],
        out_specs=tuple(bs_row(d) for d in (ATTN, 1, ATTN, ATTN, ATTN)),
        out_shape=out_shapes,
    )(geo, sem, W_geo, ln_g.reshape(1, ATTN), ln_b.reshape(1, ATTN),
      W_bdy, b_bdy.reshape(1, 1), Wq, Wk, Wv)


def _knn(coords_f, K):
    n = coords_f.shape[0]
    c2 = jnp.sum(coords_f * coords_f, axis=1)
    chunk = 2500
    out = []
    for i in range(0, n, chunk):
        q = coords_f[i:i + chunk]
        d = jnp.sum(q * q, axis=1)[:, None] + c2[None, :] - 2.0 * (q @ coords_f.T)
        _, idx = jax.lax.top_k(-d, K)
        out.append(idx)
    return jnp.concatenate(out, axis=0)


def kernel(sp_structure, geo_feat_M, sem_feat_M, W_geo, ln_g, ln_b, W_bdy,
           b_bdy, Wq, Wk, Wv, pos_emb, W_out, b_out, W_cls, b_cls):
    Q_geo, bdy_logits, Q_proj, KW, VW = _dense_precompute(
        geo_feat_M, sem_feat_M, W_geo, ln_g, ln_b, W_bdy, b_bdy, Wq, Wk, Wv)

    coords = sp_structure[:, 1:]
    neighbor_idx = _knn(coords.astype(jnp.float32), K_SEQ)
    neighbor_coords = coords[neighbor_idx]
    rel = neighbor_coords - coords[:, None, :]
    valid_mask = jnp.max(jnp.abs(rel.astype(jnp.float32)), axis=-1) <= R + 0.1
    rel_int = jnp.clip(rel + R, 0, 2 * R)
    pos_indices = rel_int[:, :, 0] * DIAM ** 2 + rel_int[:, :, 1] * DIAM + rel_int[:, :, 2]

    K_proj = KW[neighbor_idx] + pos_emb[pos_indices]
    V_proj = VW[neighbor_idx]
    attn_logits = jnp.einsum('md,mkd->mk', Q_proj, K_proj) / (ATTN ** 0.5)
    attn_logits = jnp.where(valid_mask, attn_logits, -10000.0)
    affinity = jax.nn.softmax(attn_logits, axis=-1)
    refined = jnp.einsum('mk,mkd->md', affinity, V_proj)
    refined = refined + VW
    refined_feat = refined @ W_out + b_out
    logits = refined_feat @ W_cls + b_cls
    return (logits, bdy_logits, affinity[:, None, :], refined_feat,
            neighbor_idx, valid_mask)


# speedup vs baseline: 3.6227x; 3.5950x over previous
import functools

import jax
import jax.numpy as jnp
from jax.experimental import pallas as pl
from jax.experimental.pallas import tpu as pltpu

R = 1
K_SEQ = 27
DIAM = 3
ATTN = 64
GEO = 32
SEM = 32
NCLS = 13
M = 20000


def _dense_body(geo_ref, sem_ref, Wg_ref, g_ref, b_ref, Wb_ref, bb_ref,
                Wq_ref, Wk_ref, Wv_ref,
                qgeo_ref, bdy_ref, qp_ref, kw_ref, vw_ref):
    x = geo_ref[...] @ Wg_ref[...]
    mu = jnp.mean(x, axis=-1, keepdims=True)
    var = jnp.mean((x - mu) ** 2, axis=-1, keepdims=True)
    q = (x - mu) / jnp.sqrt(var + 1e-5) * g_ref[...] + b_ref[...]
    q = jnp.maximum(q, 0.0)
    qgeo_ref[...] = q
    bdy_ref[...] = q @ Wb_ref[...] + bb_ref[...]
    qp_ref[...] = q @ Wq_ref[...]
    kw_ref[...] = q @ Wk_ref[...]
    vw_ref[...] = sem_ref[...] @ Wv_ref[...]


def _dense_precompute(geo, sem, W_geo, ln_g, ln_b, W_bdy, b_bdy, Wq, Wk, Wv):
    B = 2000
    grid = (M // B,)
    bs_row = lambda d: pl.BlockSpec((B, d), lambda i: (i, 0))
    bs_full = lambda a, b: pl.BlockSpec((a, b), lambda i: (0, 0))
    out_shapes = (
        jax.ShapeDtypeStruct((M, ATTN), jnp.float32),
        jax.ShapeDtypeStruct((M, 1), jnp.float32),
        jax.ShapeDtypeStruct((M, ATTN), jnp.float32),
        jax.ShapeDtypeStruct((M, ATTN), jnp.float32),
        jax.ShapeDtypeStruct((M, ATTN), jnp.float32),
    )
    return pl.pallas_call(
        _dense_body,
        grid=grid,
        in_specs=[
            bs_row(GEO), bs_row(SEM),
            bs_full(GEO, ATTN), bs_full(1, ATTN), bs_full(1, ATTN),
            bs_full(ATTN, 1), bs_full(1, 1),
            bs_full(ATTN, ATTN), bs_full(ATTN, ATTN), bs_full(SEM, ATTN),
        ],
        out_specs=tuple(bs_row(d) for d in (ATTN, 1, ATTN, ATTN, ATTN)),
        out_shape=out_shapes,
    )(geo, sem, W_geo, ln_g.reshape(1, ATTN), ln_b.reshape(1, ATTN),
      W_bdy, b_bdy.reshape(1, 1), Wq, Wk, Wv)


NPAD = 20096
_IMAX = 2147483647


def _knn_body(q4_ref, ct4_ref, c2_ref, out_ref):
    s = jnp.dot(q4_ref[...], ct4_ref[...], preferred_element_type=jnp.float32,
                precision=jax.lax.Precision.HIGHEST)
    d = s + c2_ref[...]
    di = jnp.minimum(d, 65535.0).astype(jnp.int32)
    col = jax.lax.broadcasted_iota(jnp.int32, (q4_ref.shape[0], NPAD), 1)
    keys = di * 32768 + col
    picked = []
    for _ in range(K_SEQ):
        m = jnp.min(keys, axis=1)
        picked.append(m)
        keys = jnp.where(keys == m[:, None], jnp.int32(_IMAX), keys)
    out_ref[...] = jnp.stack(picked, axis=1)


def _knn_pallas(coords):
    B = 200
    cf = coords.astype(jnp.float32)
    c2 = jnp.sum(cf * cf, axis=1)
    q4 = jnp.concatenate([cf, jnp.ones((M, 1), jnp.float32)], axis=1)
    big = jnp.full((NPAD - M, 3), 1.0e4, jnp.float32)
    cpad = jnp.concatenate([cf, big], axis=0)
    c2pad = jnp.sum(cpad * cpad, axis=1)
    ct4 = jnp.concatenate([-2.0 * cpad.T, c2pad[None, :]], axis=0)
    keys27 = pl.pallas_call(
        _knn_body,
        grid=(M // B,),
        in_specs=[
            pl.BlockSpec((B, 4), lambda i: (i, 0)),
            pl.BlockSpec((4, NPAD), lambda i: (0, 0)),
            pl.BlockSpec((B, 1), lambda i: (i, 0)),
        ],
        out_specs=pl.BlockSpec((B, K_SEQ), lambda i: (i, 0)),
        out_shape=jax.ShapeDtypeStruct((M, K_SEQ), jnp.int32),
    )(q4, ct4, c2[:, None])
    return keys27


def kernel(sp_structure, geo_feat_M, sem_feat_M, W_geo, ln_g, ln_b, W_bdy,
           b_bdy, Wq, Wk, Wv, pos_emb, W_out, b_out, W_cls, b_cls):
    Q_geo, bdy_logits, Q_proj, KW, VW = _dense_precompute(
        geo_feat_M, sem_feat_M, W_geo, ln_g, ln_b, W_bdy, b_bdy, Wq, Wk, Wv)

    coords = sp_structure[:, 1:]
    keys27 = _knn_pallas(coords)
    neighbor_idx = keys27 & 32767
    ndist = keys27 >> 15
    valid_mask = ndist <= 3
    neighbor_coords = coords[neighbor_idx]
    rel = neighbor_coords - coords[:, None, :]
    rel_int = jnp.clip(rel + R, 0, 2 * R)
    pos_indices = rel_int[:, :, 0] * DIAM ** 2 + rel_int[:, :, 1] * DIAM + rel_int[:, :, 2]

    K_proj = KW[neighbor_idx] + pos_emb[pos_indices]
    V_proj = VW[neighbor_idx]
    attn_logits = jnp.einsum('md,mkd->mk', Q_proj, K_proj) / (ATTN ** 0.5)
    attn_logits = jnp.where(valid_mask, attn_logits, -10000.0)
    affinity = jax.nn.softmax(attn_logits, axis=-1)
    refined = jnp.einsum('mk,mkd->md', affinity, V_proj)
    refined = refined + VW
    refined_feat = refined @ W_out + b_out
    logits = refined_feat @ W_cls + b_cls
    return (logits, bdy_logits, affinity[:, None, :], refined_feat,
            neighbor_idx, valid_mask)


# speedup vs baseline: 4.4309x; 1.2231x over previous
import functools

import jax
import jax.numpy as jnp
from jax.experimental import pallas as pl
from jax.experimental.pallas import tpu as pltpu

R = 1
K_SEQ = 27
DIAM = 3
ATTN = 64
GEO = 32
SEM = 32
NCLS = 13
M = 20000


def _dense_body(geo_ref, sem_ref, Wg_ref, g_ref, b_ref, Wb_ref, bb_ref,
                Wq_ref, Wk_ref, Wv_ref,
                qgeo_ref, bdy_ref, qp_ref, kw_ref, vw_ref):
    x = geo_ref[...] @ Wg_ref[...]
    mu = jnp.mean(x, axis=-1, keepdims=True)
    var = jnp.mean((x - mu) ** 2, axis=-1, keepdims=True)
    q = (x - mu) / jnp.sqrt(var + 1e-5) * g_ref[...] + b_ref[...]
    q = jnp.maximum(q, 0.0)
    qgeo_ref[...] = q
    bdy_ref[...] = q @ Wb_ref[...] + bb_ref[...]
    qp_ref[...] = q @ Wq_ref[...]
    kw_ref[...] = q @ Wk_ref[...]
    vw_ref[...] = sem_ref[...] @ Wv_ref[...]


def _dense_precompute(geo, sem, W_geo, ln_g, ln_b, W_bdy, b_bdy, Wq, Wk, Wv):
    B = 2000
    grid = (M // B,)
    bs_row = lambda d: pl.BlockSpec((B, d), lambda i: (i, 0))
    bs_full = lambda a, b: pl.BlockSpec((a, b), lambda i: (0, 0))
    out_shapes = (
        jax.ShapeDtypeStruct((M, ATTN), jnp.float32),
        jax.ShapeDtypeStruct((M, 1), jnp.float32),
        jax.ShapeDtypeStruct((M, ATTN), jnp.float32),
        jax.ShapeDtypeStruct((M, ATTN), jnp.float32),
        jax.ShapeDtypeStruct((M, ATTN), jnp.float32),
    )
    return pl.pallas_call(
        _dense_body,
        grid=grid,
        in_specs=[
            bs_row(GEO), bs_row(SEM),
            bs_full(GEO, ATTN), bs_full(1, ATTN), bs_full(1, ATTN),
            bs_full(ATTN, 1), bs_full(1, 1),
            bs_full(ATTN, ATTN), bs_full(ATTN, ATTN), bs_full(SEM, ATTN),
        ],
        out_specs=tuple(bs_row(d) for d in (ATTN, 1, ATTN, ATTN, ATTN)),
        out_shape=out_shapes,
    )(geo, sem, W_geo, ln_g.reshape(1, ATTN), ln_b.reshape(1, ATTN),
      W_bdy, b_bdy.reshape(1, 1), Wq, Wk, Wv)


NPAD = 20096
_IMAX = 2147483647


def _knn_body(q8_ref, c8t_ref, qc2_ref, cc2_ref, out_ref):
    B = q8_ref.shape[0]
    s = jnp.dot(q8_ref[...], c8t_ref[...], preferred_element_type=jnp.int32)
    d = qc2_ref[...] + (cc2_ref[...] - 2 * s)
    col = jax.lax.broadcasted_iota(jnp.int32, (B, NPAD), 1)
    imin = jnp.int32(-2147483648)
    rkeys = d * 32768 + col + imin
    picked = []
    prev1 = jnp.zeros((B, 1), jnp.int32)
    for i in range(K_SEQ):
        w = jnp.min(rkeys - prev1, axis=1, keepdims=True)
        m = w + imin + prev1
        picked.append(m)
        prev1 = m + 1
    out_ref[...] = jnp.concatenate(picked, axis=1)


def _knn_pallas(coords):
    B = 200
    q8 = coords.astype(jnp.int8)
    c8t = jnp.concatenate([q8.T, jnp.zeros((3, NPAD - M), jnp.int8)], axis=1)
    c2 = jnp.sum(coords * coords, axis=1)
    cc2 = jnp.concatenate([c2, jnp.full((NPAD - M,), 30000, jnp.int32)])
    keys27 = pl.pallas_call(
        _knn_body,
        grid=(M // B,),
        in_specs=[
            pl.BlockSpec((B, 3), lambda i: (i, 0)),
            pl.BlockSpec((3, NPAD), lambda i: (0, 0)),
            pl.BlockSpec((B, 1), lambda i: (i, 0)),
            pl.BlockSpec((1, NPAD), lambda i: (0, 0)),
        ],
        out_specs=pl.BlockSpec((B, K_SEQ), lambda i: (i, 0)),
        out_shape=jax.ShapeDtypeStruct((M, K_SEQ), jnp.int32),
    )(q8, c8t, c2[:, None], cc2[None, :])
    return keys27


def kernel(sp_structure, geo_feat_M, sem_feat_M, W_geo, ln_g, ln_b, W_bdy,
           b_bdy, Wq, Wk, Wv, pos_emb, W_out, b_out, W_cls, b_cls):
    Q_geo, bdy_logits, Q_proj, KW, VW = _dense_precompute(
        geo_feat_M, sem_feat_M, W_geo, ln_g, ln_b, W_bdy, b_bdy, Wq, Wk, Wv)

    coords = sp_structure[:, 1:]
    keys27 = _knn_pallas(coords)
    neighbor_idx = keys27 & 32767
    ndist = keys27 >> 15
    valid_mask = ndist <= 3
    neighbor_coords = coords[neighbor_idx]
    rel = neighbor_coords - coords[:, None, :]
    rel_int = jnp.clip(rel + R, 0, 2 * R)
    pos_indices = rel_int[:, :, 0] * DIAM ** 2 + rel_int[:, :, 1] * DIAM + rel_int[:, :, 2]

    K_proj = KW[neighbor_idx] + pos_emb[pos_indices]
    V_proj = VW[neighbor_idx]
    attn_logits = jnp.einsum('md,mkd->mk', Q_proj, K_proj) / (ATTN ** 0.5)
    attn_logits = jnp.where(valid_mask, attn_logits, -10000.0)
    affinity = jax.nn.softmax(attn_logits, axis=-1)
    refined = jnp.einsum('mk,mkd->md', affinity, V_proj)
    refined = refined + VW
    refined_feat = refined @ W_out + b_out
    logits = refined_feat @ W_cls + b_cls
    return (logits, bdy_logits, affinity[:, None, :], refined_feat,
            neighbor_idx, valid_mask)


# speedup vs baseline: 6.6216x; 1.4944x over previous
import functools

import jax
import jax.numpy as jnp
from jax.experimental import pallas as pl
from jax.experimental.pallas import tpu as pltpu

R = 1
K_SEQ = 27
DIAM = 3
ATTN = 64
GEO = 32
SEM = 32
NCLS = 13
M = 20000


def _dense_body(geo_ref, sem_ref, Wg_ref, g_ref, b_ref, Wb_ref, bb_ref,
                Wq_ref, Wk_ref, Wv_ref,
                qgeo_ref, bdy_ref, qp_ref, kw_ref, vw_ref):
    x = geo_ref[...] @ Wg_ref[...]
    mu = jnp.mean(x, axis=-1, keepdims=True)
    var = jnp.mean((x - mu) ** 2, axis=-1, keepdims=True)
    q = (x - mu) / jnp.sqrt(var + 1e-5) * g_ref[...] + b_ref[...]
    q = jnp.maximum(q, 0.0)
    qgeo_ref[...] = q
    bdy_ref[...] = q @ Wb_ref[...] + bb_ref[...]
    qp_ref[...] = q @ Wq_ref[...]
    kw_ref[...] = q @ Wk_ref[...]
    vw_ref[...] = sem_ref[...] @ Wv_ref[...]


def _dense_precompute(geo, sem, W_geo, ln_g, ln_b, W_bdy, b_bdy, Wq, Wk, Wv):
    B = 2000
    grid = (M // B,)
    bs_row = lambda d: pl.BlockSpec((B, d), lambda i: (i, 0))
    bs_full = lambda a, b: pl.BlockSpec((a, b), lambda i: (0, 0))
    out_shapes = (
        jax.ShapeDtypeStruct((M, ATTN), jnp.float32),
        jax.ShapeDtypeStruct((M, 1), jnp.float32),
        jax.ShapeDtypeStruct((M, ATTN), jnp.float32),
        jax.ShapeDtypeStruct((M, ATTN), jnp.float32),
        jax.ShapeDtypeStruct((M, ATTN), jnp.float32),
    )
    return pl.pallas_call(
        _dense_body,
        grid=grid,
        in_specs=[
            bs_row(GEO), bs_row(SEM),
            bs_full(GEO, ATTN), bs_full(1, ATTN), bs_full(1, ATTN),
            bs_full(ATTN, 1), bs_full(1, 1),
            bs_full(ATTN, ATTN), bs_full(ATTN, ATTN), bs_full(SEM, ATTN),
        ],
        out_specs=tuple(bs_row(d) for d in (ATTN, 1, ATTN, ATTN, ATTN)),
        out_shape=out_shapes,
    )(geo, sem, W_geo, ln_g.reshape(1, ATTN), ln_b.reshape(1, ATTN),
      W_bdy, b_bdy.reshape(1, 1), Wq, Wk, Wv)


NPAD = 20096
LEVELS = 5
_IMAX = 2147483647


def _knn_body(q8_ref, c8t_ref, qc2_ref, cc2_ref, out_ref):
    B = q8_ref.shape[0]
    s = jnp.dot(q8_ref[...], c8t_ref[...], preferred_element_type=jnp.int32)
    d = qc2_ref[...] + (cc2_ref[...] - 2 * s)
    col = jax.lax.broadcasted_iota(jnp.int32, (B, NPAD), 1)
    imin = jnp.int32(-2147483648)
    rkeys = d * 32768 + col + imin
    rk3 = rkeys.reshape(B, NPAD // 128, 128)

    levels = []
    p_lane = jnp.zeros((B, 1, 128), jnp.int32)
    for _ in range(LEVELS):
        w = jnp.min(rk3 - p_lane, axis=1, keepdims=True)
        levels.append(w + p_lane)
        p_lane = w + imin + p_lane + 1
    cand = jnp.concatenate(levels, axis=1).reshape(B, LEVELS * 128)

    picked = []
    prev1 = jnp.zeros((B, 1), jnp.int32)
    for _ in range(K_SEQ):
        w = jnp.min(cand - prev1, axis=1, keepdims=True)
        picked.append(w + imin + prev1)
        prev1 = w + imin + prev1 + 1
    fast = jnp.concatenate(picked, axis=1)

    k27_rot = picked[-1] + imin
    suspect = jnp.any(levels[-1][:, 0, :] <= k27_rot)

    @pl.when(jnp.logical_not(suspect))
    def _():
        out_ref[...] = fast

    @pl.when(suspect)
    def _():
        slow = []
        p1 = jnp.zeros((B, 1), jnp.int32)
        for _ in range(K_SEQ):
            w = jnp.min(rkeys - p1, axis=1, keepdims=True)
            slow.append(w + imin + p1)
            p1 = w + imin + p1 + 1
        out_ref[...] = jnp.concatenate(slow, axis=1)


def _knn_pallas(coords):
    B = 200
    q8 = coords.astype(jnp.int8)
    c8t = jnp.concatenate([q8.T, jnp.zeros((3, NPAD - M), jnp.int8)], axis=1)
    c2 = jnp.sum(coords * coords, axis=1)
    cc2 = jnp.concatenate([c2, jnp.full((NPAD - M,), 30000, jnp.int32)])
    keys27 = pl.pallas_call(
        _knn_body,
        grid=(M // B,),
        in_specs=[
            pl.BlockSpec((B, 3), lambda i: (i, 0)),
            pl.BlockSpec((3, NPAD), lambda i: (0, 0)),
            pl.BlockSpec((B, 1), lambda i: (i, 0)),
            pl.BlockSpec((1, NPAD), lambda i: (0, 0)),
        ],
        out_specs=pl.BlockSpec((B, K_SEQ), lambda i: (i, 0)),
        out_shape=jax.ShapeDtypeStruct((M, K_SEQ), jnp.int32),
    )(q8, c8t, c2[:, None], cc2[None, :])
    return keys27


def kernel(sp_structure, geo_feat_M, sem_feat_M, W_geo, ln_g, ln_b, W_bdy,
           b_bdy, Wq, Wk, Wv, pos_emb, W_out, b_out, W_cls, b_cls):
    Q_geo, bdy_logits, Q_proj, KW, VW = _dense_precompute(
        geo_feat_M, sem_feat_M, W_geo, ln_g, ln_b, W_bdy, b_bdy, Wq, Wk, Wv)

    coords = sp_structure[:, 1:]
    keys27 = _knn_pallas(coords)
    neighbor_idx = keys27 & 32767
    ndist = keys27 >> 15
    valid_mask = ndist <= 3
    neighbor_coords = coords[neighbor_idx]
    rel = neighbor_coords - coords[:, None, :]
    rel_int = jnp.clip(rel + R, 0, 2 * R)
    pos_indices = rel_int[:, :, 0] * DIAM ** 2 + rel_int[:, :, 1] * DIAM + rel_int[:, :, 2]

    K_proj = KW[neighbor_idx] + pos_emb[pos_indices]
    V_proj = VW[neighbor_idx]
    attn_logits = jnp.einsum('md,mkd->mk', Q_proj, K_proj) / (ATTN ** 0.5)
    attn_logits = jnp.where(valid_mask, attn_logits, -10000.0)
    affinity = jax.nn.softmax(attn_logits, axis=-1)
    refined = jnp.einsum('mk,mkd->md', affinity, V_proj)
    refined = refined + VW
    refined_feat = refined @ W_out + b_out
    logits = refined_feat @ W_cls + b_cls
    return (logits, bdy_logits, affinity[:, None, :], refined_feat,
            neighbor_idx, valid_mask)


# speedup vs baseline: 7.1765x; 1.0838x over previous
import functools

import jax
import jax.numpy as jnp
from jax.experimental import pallas as pl
from jax.experimental.pallas import tpu as pltpu

R = 1
K_SEQ = 27
DIAM = 3
ATTN = 64
GEO = 32
SEM = 32
NCLS = 13
M = 20000


def _dense_body(geo_ref, sem_ref, Wg_ref, g_ref, b_ref, Wb_ref, bb_ref,
                Wq_ref, Wk_ref, Wv_ref,
                qgeo_ref, bdy_ref, qp_ref, kw_ref, vw_ref):
    x = geo_ref[...] @ Wg_ref[...]
    mu = jnp.mean(x, axis=-1, keepdims=True)
    var = jnp.mean((x - mu) ** 2, axis=-1, keepdims=True)
    q = (x - mu) / jnp.sqrt(var + 1e-5) * g_ref[...] + b_ref[...]
    q = jnp.maximum(q, 0.0)
    qgeo_ref[...] = q
    bdy_ref[...] = q @ Wb_ref[...] + bb_ref[...]
    qp_ref[...] = q @ Wq_ref[...]
    kw_ref[...] = q @ Wk_ref[...]
    vw_ref[...] = sem_ref[...] @ Wv_ref[...]


def _dense_precompute(geo, sem, W_geo, ln_g, ln_b, W_bdy, b_bdy, Wq, Wk, Wv):
    B = 2000
    grid = (M // B,)
    bs_row = lambda d: pl.BlockSpec((B, d), lambda i: (i, 0))
    bs_full = lambda a, b: pl.BlockSpec((a, b), lambda i: (0, 0))
    out_shapes = (
        jax.ShapeDtypeStruct((M, ATTN), jnp.float32),
        jax.ShapeDtypeStruct((M, 1), jnp.float32),
        jax.ShapeDtypeStruct((M, ATTN), jnp.float32),
        jax.ShapeDtypeStruct((M, ATTN), jnp.float32),
        jax.ShapeDtypeStruct((M, ATTN), jnp.float32),
    )
    return pl.pallas_call(
        _dense_body,
        grid=grid,
        in_specs=[
            bs_row(GEO), bs_row(SEM),
            bs_full(GEO, ATTN), bs_full(1, ATTN), bs_full(1, ATTN),
            bs_full(ATTN, 1), bs_full(1, 1),
            bs_full(ATTN, ATTN), bs_full(ATTN, ATTN), bs_full(SEM, ATTN),
        ],
        out_specs=tuple(bs_row(d) for d in (ATTN, 1, ATTN, ATTN, ATTN)),
        out_shape=out_shapes,
    )(geo, sem, W_geo, ln_g.reshape(1, ATTN), ln_b.reshape(1, ATTN),
      W_bdy, b_bdy.reshape(1, 1), Wq, Wk, Wv)


NPAD = 20480
LEVELS = 5
_IMAX = 2147483647


def _knn_body(q8_ref, c8t_ref, qc2_ref, cc2_ref, out_ref):
    B = q8_ref.shape[0]
    s = jnp.dot(q8_ref[...], c8t_ref[...], preferred_element_type=jnp.int32)
    d = qc2_ref[...] + (cc2_ref[...] - 2 * s)
    col = jax.lax.broadcasted_iota(jnp.int32, (B, NPAD), 1)
    imin = jnp.int32(-2147483648)
    rkeys = d * 32768 + col + imin
    nslice = NPAD // 128

    levels = []
    p_lane = jnp.zeros((B, 128), jnp.int32)
    for lv in range(LEVELS):
        if lv == 0:
            acc = rkeys[:, 0:128]
            for k in range(1, nslice):
                acc = jnp.minimum(acc, rkeys[:, k * 128:(k + 1) * 128])
        else:
            acc = rkeys[:, 0:128] - p_lane
            for k in range(1, nslice):
                acc = jnp.minimum(acc, rkeys[:, k * 128:(k + 1) * 128] - p_lane)
        levels.append(acc + p_lane)
        p_lane = acc + imin + p_lane + 1
    cand = jnp.concatenate(levels, axis=1)

    picked = []
    prev1 = jnp.zeros((B, 1), jnp.int32)
    for _ in range(K_SEQ):
        w = jnp.min(cand - prev1, axis=1, keepdims=True)
        picked.append(w + imin + prev1)
        prev1 = w + imin + prev1 + 1
    fast = jnp.concatenate(picked, axis=1)

    k27_rot = picked[-1] + imin
    suspect = jnp.any(levels[-1] <= k27_rot)

    @pl.when(jnp.logical_not(suspect))
    def _():
        out_ref[...] = fast

    @pl.when(suspect)
    def _():
        slow = []
        p1 = jnp.zeros((B, 1), jnp.int32)
        for _ in range(K_SEQ):
            w = jnp.min(rkeys - p1, axis=1, keepdims=True)
            slow.append(w + imin + p1)
            p1 = w + imin + p1 + 1
        out_ref[...] = jnp.concatenate(slow, axis=1)


def _knn_pallas(coords):
    B = 200
    q8 = coords.astype(jnp.int8)
    c8t = jnp.concatenate([q8.T, jnp.zeros((3, NPAD - M), jnp.int8)], axis=1)
    c2 = jnp.sum(coords * coords, axis=1)
    cc2 = jnp.concatenate([c2, jnp.full((NPAD - M,), 30000, jnp.int32)])
    keys27 = pl.pallas_call(
        _knn_body,
        grid=(M // B,),
        in_specs=[
            pl.BlockSpec((B, 3), lambda i: (i, 0)),
            pl.BlockSpec((3, NPAD), lambda i: (0, 0)),
            pl.BlockSpec((B, 1), lambda i: (i, 0)),
            pl.BlockSpec((1, NPAD), lambda i: (0, 0)),
        ],
        out_specs=pl.BlockSpec((B, K_SEQ), lambda i: (i, 0)),
        out_shape=jax.ShapeDtypeStruct((M, K_SEQ), jnp.int32),
    )(q8, c8t, c2[:, None], cc2[None, :])
    return keys27


def kernel(sp_structure, geo_feat_M, sem_feat_M, W_geo, ln_g, ln_b, W_bdy,
           b_bdy, Wq, Wk, Wv, pos_emb, W_out, b_out, W_cls, b_cls):
    Q_geo, bdy_logits, Q_proj, KW, VW = _dense_precompute(
        geo_feat_M, sem_feat_M, W_geo, ln_g, ln_b, W_bdy, b_bdy, Wq, Wk, Wv)

    coords = sp_structure[:, 1:]
    keys27 = _knn_pallas(coords)
    neighbor_idx = keys27 & 32767
    ndist = keys27 >> 15
    valid_mask = ndist <= 3
    neighbor_coords = coords[neighbor_idx]
    rel = neighbor_coords - coords[:, None, :]
    rel_int = jnp.clip(rel + R, 0, 2 * R)
    pos_indices = rel_int[:, :, 0] * DIAM ** 2 + rel_int[:, :, 1] * DIAM + rel_int[:, :, 2]

    K_proj = KW[neighbor_idx] + pos_emb[pos_indices]
    V_proj = VW[neighbor_idx]
    attn_logits = jnp.einsum('md,mkd->mk', Q_proj, K_proj) / (ATTN ** 0.5)
    attn_logits = jnp.where(valid_mask, attn_logits, -10000.0)
    affinity = jax.nn.softmax(attn_logits, axis=-1)
    refined = jnp.einsum('mk,mkd->md', affinity, V_proj)
    refined = refined + VW
    refined_feat = refined @ W_out + b_out
    logits = refined_feat @ W_cls + b_cls
    return (logits, bdy_logits, affinity[:, None, :], refined_feat,
            neighbor_idx, valid_mask)
